# baseline (speedup 1.0000x reference)
import functools

import jax
import jax.numpy as jnp
from jax import lax
from jax.experimental import pallas as pl
from jax.experimental.pallas import tpu as pltpu
from jax.experimental.pallas import tpu_sc as plsc

_D_MODEL = 128
_MAX_TEMPO = 300
_BATCH = 16384
_NC = 2
_NS = 16
_NW = _NC * _NS
_B_PER_W = _BATCH // _NW
_CH = 128
_NCHUNK = _B_PER_W // _CH

_mesh = plsc.VectorSubcoreMesh(core_axis_name="c", subcore_axis_name="s")


@functools.partial(
    pl.kernel,
    mesh=_mesh,
    out_type=jax.ShapeDtypeStruct((_BATCH, _D_MODEL), jnp.float32),
    scratch_types=[
        pltpu.VMEM((_B_PER_W,), jnp.int32),
        pltpu.VMEM((_B_PER_W, _D_MODEL), jnp.float32),
        pltpu.VMEM_SHARED((_MAX_TEMPO, _D_MODEL), jnp.float32),
    ]
    + [pltpu.SemaphoreType.DMA] * (2 * _NCHUNK),
)
def _gather_kernel(tempo_hbm, pe_hbm, out_hbm, idx_v, rows_v, table_s, *sems):
    gsems, ssems = sems[:_NCHUNK], sems[_NCHUNK:]
    cid = lax.axis_index("c")
    sid = lax.axis_index("s")
    wid = sid * _NC + cid
    base = wid * _B_PER_W
    pltpu.sync_copy(tempo_hbm.at[pl.ds(base, _B_PER_W)], idx_v)

    @pl.when(sid == 0)
    def _stage_table():
        pltpu.sync_copy(pe_hbm, table_s)

    plsc.subcore_barrier()
    gathers = [
        pltpu.async_copy(
            table_s.at[idx_v.at[pl.ds(j * _CH, _CH)]],
            rows_v.at[pl.ds(j * _CH, _CH)],
            gsems[j],
        )
        for j in range(_NCHUNK)
    ]
    stores = []
    for j in range(_NCHUNK):
        gathers[j].wait()
        stores.append(
            pltpu.async_copy(
                rows_v.at[pl.ds(j * _CH, _CH)],
                out_hbm.at[pl.ds(base + j * _CH, _CH)],
                ssems[j],
            )
        )
    for s in stores:
        s.wait()


def kernel(tempo, pe):
    return _gather_kernel(tempo, pe)


# speedup vs baseline: 1.0148x; 1.0148x over previous
import functools

import jax
import jax.numpy as jnp
from jax import lax
from jax.experimental import pallas as pl
from jax.experimental.pallas import tpu as pltpu
from jax.experimental.pallas import tpu_sc as plsc

_D_MODEL = 128
_MAX_TEMPO = 300
_BATCH = 16384
_NC = 2
_NS = 16
_NW = _NC * _NS
_B_PER_W = _BATCH // _NW
_CH = 64
_NCHUNK = _B_PER_W // _CH

_mesh = plsc.VectorSubcoreMesh(core_axis_name="c", subcore_axis_name="s")


@functools.partial(
    pl.kernel,
    mesh=_mesh,
    out_type=jax.ShapeDtypeStruct((_BATCH, _D_MODEL), jnp.float32),
    scratch_types=[
        pltpu.VMEM((_B_PER_W,), jnp.int32),
        pltpu.VMEM((_B_PER_W, _D_MODEL), jnp.float32),
        pltpu.VMEM_SHARED((_MAX_TEMPO, _D_MODEL), jnp.float32),
    ]
    + [pltpu.SemaphoreType.DMA] * (2 * _NCHUNK),
)
def _gather_kernel(tempo_hbm, pe_hbm, out_hbm, idx_v, rows_v, table_s, *sems):
    gsems, ssems = sems[:_NCHUNK], sems[_NCHUNK:]
    cid = lax.axis_index("c")
    sid = lax.axis_index("s")
    wid = sid * _NC + cid
    base = wid * _B_PER_W
    idx_cp = pltpu.async_copy(
        tempo_hbm.at[pl.ds(base, _B_PER_W)], idx_v, ssems[0]
    )

    @pl.when(sid == 0)
    def _stage_table():
        pltpu.sync_copy(pe_hbm, table_s)

    idx_cp.wait()
    plsc.subcore_barrier()
    gathers = [
        pltpu.async_copy(
            table_s.at[idx_v.at[pl.ds(j * _CH, _CH)]],
            rows_v.at[pl.ds(j * _CH, _CH)],
            gsems[j],
        )
        for j in range(_NCHUNK)
    ]
    stores = []
    for j in range(_NCHUNK):
        gathers[j].wait()
        stores.append(
            pltpu.async_copy(
                rows_v.at[pl.ds(j * _CH, _CH)],
                out_hbm.at[pl.ds(base + j * _CH, _CH)],
                ssems[j],
            )
        )
    for s in stores:
        s.wait()


def kernel(tempo, pe):
    return _gather_kernel(tempo, pe)


# speedup vs baseline: 1.0317x; 1.0166x over previous
import functools

import jax
import jax.numpy as jnp
from jax import lax
from jax.experimental import pallas as pl
from jax.experimental.pallas import tpu as pltpu
from jax.experimental.pallas import tpu_sc as plsc

_D_MODEL = 128
_MAX_TEMPO = 300
_BATCH = 16384
_NC = 2
_NS = 16
_NW = _NC * _NS
_B_PER_W = _BATCH // _NW
_CH = 64
_NCHUNK = _B_PER_W // _CH

_mesh = plsc.VectorSubcoreMesh(core_axis_name="c", subcore_axis_name="s")


@functools.partial(
    pl.kernel,
    mesh=_mesh,
    out_type=jax.ShapeDtypeStruct((_BATCH, _D_MODEL), jnp.float32),
    scratch_types=[
        pltpu.VMEM((_B_PER_W,), jnp.int32),
        pltpu.VMEM((_B_PER_W, _D_MODEL), jnp.float32),
        pltpu.VMEM_SHARED((_MAX_TEMPO, _D_MODEL), jnp.float32),
    ]
    + [pltpu.SemaphoreType.DMA] * 2,
)
def _gather_kernel(tempo_hbm, pe_hbm, out_hbm, idx_v, rows_v, table_s, gsem, ssem):
    cid = lax.axis_index("c")
    sid = lax.axis_index("s")
    wid = sid * _NC + cid
    base = wid * _B_PER_W
    idx_cp = pltpu.async_copy(tempo_hbm.at[pl.ds(base, _B_PER_W)], idx_v, ssem)

    @pl.when(sid == 0)
    def _stage_table():
        pltpu.sync_copy(pe_hbm, table_s)

    idx_cp.wait()
    plsc.subcore_barrier()
    gathers = [
        pltpu.async_copy(
            table_s.at[idx_v.at[pl.ds(j * _CH, _CH)]],
            rows_v.at[pl.ds(j * _CH, _CH)],
            gsem,
        )
        for j in range(_NCHUNK)
    ]
    stores = []
    for j in range(_NCHUNK):
        gathers[j].wait()
        stores.append(
            pltpu.async_copy(
                rows_v.at[pl.ds(j * _CH, _CH)],
                out_hbm.at[pl.ds(base + j * _CH, _CH)],
                ssem,
            )
        )
    for s in stores:
        s.wait()


def kernel(tempo, pe):
    return _gather_kernel(tempo, pe)
